# baseline (speedup 1.0000x reference)
import functools

import jax
import jax.numpy as jnp
from jax import lax
from jax.experimental import pallas as pl
from jax.experimental.pallas import tpu as pltpu
from jax.experimental.pallas import tpu_sc as plsc

D = 64
SCALE = 8.0
NC, NS, L = 2, 16, 16
NW = NC * NS
TB = 128
VB = 4096
BT = 512

_HIGH = jax.lax.Precision.HIGHEST


def _prep_table(V):
    def body(tt_ref, tabs_ref):
        e = jnp.where(
            lax.broadcasted_iota(jnp.int32, (D, 2 * D), 0)
            == lax.broadcasted_iota(jnp.int32, (D, 2 * D), 1),
            SCALE, 0.0)
        tabs_ref[...] = lax.dot_general(
            tt_ref[...], e, (((0,), (0,)), ((), ())),
            precision=_HIGH, preferred_element_type=jnp.float32)

    return pl.pallas_call(
        body,
        grid=(pl.cdiv(V, VB),),
        in_specs=[pl.BlockSpec((D, VB), lambda v: (0, v))],
        out_specs=pl.BlockSpec((VB, 2 * D), lambda v: (v, 0)),
        out_shape=jax.ShapeDtypeStruct((V, 2 * D), jnp.float32),
    )


def _make_gather(T, S):
    n_tb = T // (NW * TB)
    n_units = S * n_tb
    assert n_units % 2 == 0 and n_units >= 4
    mesh = plsc.VectorSubcoreMesh(core_axis_name="c", subcore_axis_name="s")

    @functools.partial(
        pl.kernel,
        mesh=mesh,
        compiler_params=pltpu.CompilerParams(needs_layout_passes=False),
        out_type=jax.ShapeDtypeStruct((S, T, 2 * D), jnp.float32),
        scratch_types=[
            pltpu.VMEM((TB,), jnp.int32),
            pltpu.VMEM((TB,), jnp.int32),
            pltpu.VMEM((TB, 2 * D), jnp.float32),
            pltpu.VMEM((TB, 2 * D), jnp.float32),
            pltpu.SemaphoreType.DMA,
            pltpu.SemaphoreType.DMA,
            pltpu.SemaphoreType.DMA,
            pltpu.SemaphoreType.DMA,
        ],
    )
    def k(tok_t, tabs, mid, tokA, tokB, rowsA, rowsB, tsA, tsB, gsA, gsB):
        wid = lax.axis_index("s") * NC + lax.axis_index("c")
        t_base = wid * (n_tb * TB)

        def tok_src(u):
            s = u // n_tb
            t0 = t_base + (u % n_tb) * TB
            return tok_t.at[s, pl.ds(t0, TB)]

        def start_tok(u, tok_v, sem):
            pltpu.async_copy(tok_src(u), tok_v, sem)

        def wait_tok(u, tok_v, sem):
            pltpu.make_async_copy(tok_src(u), tok_v, sem).wait()

        def start_gather(idx_v, rows_v, sem):
            pltpu.async_copy(tabs.at[idx_v], rows_v, sem)

        def wait_gather(idx_v, rows_v, sem):
            pltpu.make_async_copy(tabs.at[idx_v], rows_v, sem).wait()

        def emit_unit(u, rows_v):
            s = u // n_tb
            t0 = t_base + (u % n_tb) * TB
            pltpu.sync_copy(rows_v, mid.at[s, pl.ds(t0, TB), :])

        pltpu.sync_copy(tok_src(0), tokA)
        start_gather(tokA, rowsA, gsA)
        start_tok(1, tokB, tsB)

        def pair(kk, carry):
            u0 = 2 * kk
            wait_tok(u0 + 1, tokB, tsB)
            wait_gather(tokA, rowsA, gsA)
            start_gather(tokB, rowsB, gsB)
            start_tok(u0 + 2, tokA, tsA)
            emit_unit(u0, rowsA)
            wait_tok(u0 + 2, tokA, tsA)
            wait_gather(tokB, rowsB, gsB)
            start_gather(tokA, rowsA, gsA)
            start_tok(u0 + 3, tokB, tsB)
            emit_unit(u0 + 1, rowsB)
            return carry

        lax.fori_loop(0, n_units // 2 - 1, pair, 0)

        wait_tok(n_units - 1, tokB, tsB)
        wait_gather(tokA, rowsA, gsA)
        start_gather(tokB, rowsB, gsB)
        emit_unit(n_units - 2, rowsA)
        wait_gather(tokB, rowsB, gsB)
        emit_unit(n_units - 1, rowsB)

    return k


def _out_transpose(S, T):
    def body(mid_ref, out_ref):
        e = jnp.where(
            lax.broadcasted_iota(jnp.int32, (BT, BT), 0)
            == lax.broadcasted_iota(jnp.int32, (BT, BT), 1),
            1.0, 0.0)
        m = mid_ref[0][:, :D]
        out_ref[0] = lax.dot_general(
            m, e, (((0,), (0,)), ((), ())),
            precision=_HIGH, preferred_element_type=jnp.float32)

    return pl.pallas_call(
        body,
        grid=(S, T // BT),
        in_specs=[pl.BlockSpec((1, BT, 2 * D), lambda s, tb: (s, tb, 0))],
        out_specs=pl.BlockSpec((1, D, BT), lambda s, tb: (s, 0, tb)),
        out_shape=jax.ShapeDtypeStruct((S, D, T), jnp.float32),
    )


def kernel(tokens, table):
    T, S = tokens.shape
    V = table.shape[0]
    tabs = _prep_table(V)(table.T)
    mid = _make_gather(T, S)(tokens.T, tabs)
    out_t = _out_transpose(S, T)(mid)
    return out_t.transpose(2, 0, 1)


# speedup vs baseline: 1.1691x; 1.1691x over previous
import functools

import jax
import jax.numpy as jnp
from jax import lax
from jax.experimental import pallas as pl
from jax.experimental.pallas import tpu as pltpu
from jax.experimental.pallas import tpu_sc as plsc

D = 64
SCALE = 8.0
NC, NS, L = 2, 16, 16
NW = NC * NS
TB = 128
VB = 4096
BT = 512

_HIGH = jax.lax.Precision.HIGHEST


def _prep_table(V):
    def body(tt_ref, tabs_ref):
        e = jnp.where(
            lax.broadcasted_iota(jnp.int32, (D, 2 * D), 0)
            == lax.broadcasted_iota(jnp.int32, (D, 2 * D), 1),
            SCALE, 0.0)
        tabs_ref[...] = lax.dot_general(
            tt_ref[...], e, (((0,), (0,)), ((), ())),
            precision=_HIGH, preferred_element_type=jnp.float32)

    return pl.pallas_call(
        body,
        grid=(pl.cdiv(V, VB),),
        in_specs=[pl.BlockSpec((D, VB), lambda v: (0, v))],
        out_specs=pl.BlockSpec((VB, 2 * D), lambda v: (v, 0)),
        out_shape=jax.ShapeDtypeStruct((V, 2 * D), jnp.float32),
    )


def _make_gather(T, S):
    n_tb = T // (NW * TB)
    n_units = S * n_tb
    assert n_units % 2 == 0 and n_units >= 4
    mesh = plsc.VectorSubcoreMesh(core_axis_name="c", subcore_axis_name="s")

    @functools.partial(
        pl.kernel,
        mesh=mesh,
        compiler_params=pltpu.CompilerParams(needs_layout_passes=False),
        out_type=jax.ShapeDtypeStruct((S, T, 2 * D), jnp.float32),
        scratch_types=[
            pltpu.VMEM((TB,), jnp.int32),
            pltpu.VMEM((TB,), jnp.int32),
            pltpu.VMEM((TB, 2 * D), jnp.float32),
            pltpu.VMEM((TB, 2 * D), jnp.float32),
            pltpu.SemaphoreType.DMA,
            pltpu.SemaphoreType.DMA,
            pltpu.SemaphoreType.DMA,
            pltpu.SemaphoreType.DMA,
        ],
    )
    def k(tok_t, tabs, mid, tokA, tokB, rowsA, rowsB, tsA, tsB, gsA, gsB):
        wid = lax.axis_index("s") * NC + lax.axis_index("c")
        t_base = wid * (n_tb * TB)

        def tok_src(u):
            s = u // n_tb
            t0 = t_base + (u % n_tb) * TB
            return tok_t.at[s, pl.ds(t0, TB)]

        def start_tok(u, tok_v, sem):
            pltpu.async_copy(tok_src(u), tok_v, sem)

        def wait_tok(u, tok_v, sem):
            pltpu.make_async_copy(tok_src(u), tok_v, sem).wait()

        def start_gather(idx_v, rows_v, sem):
            pltpu.async_copy(tabs.at[idx_v], rows_v, sem)

        def wait_gather(idx_v, rows_v, sem):
            pltpu.make_async_copy(tabs.at[idx_v], rows_v, sem).wait()

        def emit_unit(u, rows_v):
            s = u // n_tb
            t0 = t_base + (u % n_tb) * TB
            pltpu.sync_copy(rows_v, mid.at[s, pl.ds(t0, TB), :])

        pltpu.sync_copy(tok_src(0), tokA)
        start_gather(tokA, rowsA, gsA)
        start_tok(1, tokB, tsB)

        def pair(kk, carry):
            u0 = 2 * kk
            wait_tok(u0 + 1, tokB, tsB)
            wait_gather(tokA, rowsA, gsA)
            start_gather(tokB, rowsB, gsB)
            start_tok(u0 + 2, tokA, tsA)
            emit_unit(u0, rowsA)
            wait_tok(u0 + 2, tokA, tsA)
            wait_gather(tokB, rowsB, gsB)
            start_gather(tokA, rowsA, gsA)
            start_tok(u0 + 3, tokB, tsB)
            emit_unit(u0 + 1, rowsB)
            return carry

        lax.fori_loop(0, n_units // 2 - 1, pair, 0)

        wait_tok(n_units - 1, tokB, tsB)
        wait_gather(tokA, rowsA, gsA)
        start_gather(tokB, rowsB, gsB)
        emit_unit(n_units - 2, rowsA)
        wait_gather(tokB, rowsB, gsB)
        emit_unit(n_units - 1, rowsB)

    return k


def _out_transpose(S, T):
    def body(mid_ref, out_ref):
        m = mid_ref[0][:, :D]
        out_ref[0] = jnp.transpose(m, (1, 0))

    return pl.pallas_call(
        body,
        grid=(S, T // BT),
        in_specs=[pl.BlockSpec((1, BT, 2 * D), lambda s, tb: (s, tb, 0))],
        out_specs=pl.BlockSpec((1, D, BT), lambda s, tb: (s, 0, tb)),
        out_shape=jax.ShapeDtypeStruct((S, D, T), jnp.float32),
    )


def kernel(tokens, table):
    T, S = tokens.shape
    V = table.shape[0]
    tabs = _prep_table(V)(table.T)
    mid = _make_gather(T, S)(tokens.T, tabs)
    out_t = _out_transpose(S, T)(mid)
    return out_t.transpose(2, 0, 1)


# speedup vs baseline: 2.0966x; 1.7933x over previous
import functools

import jax
import jax.numpy as jnp
from jax import lax
from jax.experimental import pallas as pl
from jax.experimental.pallas import tpu as pltpu
from jax.experimental.pallas import tpu_sc as plsc

D = 64
SCALE = 8.0
NC, NS, L = 2, 16, 16
NW = NC * NS
TB = 128
VB = 4096
BT = 512

_HIGH = jax.lax.Precision.HIGHEST


def _prep_table(V):
    def body(tt_ref, tabs_ref):
        e = jnp.where(
            lax.broadcasted_iota(jnp.int32, (D, 2 * D), 0)
            == lax.broadcasted_iota(jnp.int32, (D, 2 * D), 1),
            SCALE, 0.0)
        tabs_ref[...] = lax.dot_general(
            tt_ref[...], e, (((0,), (0,)), ((), ())),
            precision=_HIGH, preferred_element_type=jnp.float32)

    return pl.pallas_call(
        body,
        grid=(pl.cdiv(V, VB),),
        in_specs=[pl.BlockSpec((D, VB), lambda v: (0, v))],
        out_specs=pl.BlockSpec((VB, 2 * D), lambda v: (v, 0)),
        out_shape=jax.ShapeDtypeStruct((V, 2 * D), jnp.float32),
    )


def _make_gather(T, S):
    n_tb = T // (NW * TB)
    n_units = S * n_tb
    assert n_units % 2 == 0 and n_units >= 4
    mesh = plsc.VectorSubcoreMesh(core_axis_name="c", subcore_axis_name="s")

    @functools.partial(
        pl.kernel,
        mesh=mesh,
        compiler_params=pltpu.CompilerParams(needs_layout_passes=False),
        out_type=jax.ShapeDtypeStruct((T, S, 2 * D), jnp.float32),
        scratch_types=[
            pltpu.VMEM((TB,), jnp.int32),
            pltpu.VMEM((TB,), jnp.int32),
            pltpu.VMEM((TB, 2 * D), jnp.float32),
            pltpu.VMEM((TB, 2 * D), jnp.float32),
            pltpu.SemaphoreType.DMA,
            pltpu.SemaphoreType.DMA,
            pltpu.SemaphoreType.DMA,
            pltpu.SemaphoreType.DMA,
        ],
    )
    def k(tok_t, tabs, mid, tokA, tokB, rowsA, rowsB, tsA, tsB, gsA, gsB):
        wid = lax.axis_index("s") * NC + lax.axis_index("c")
        t_base = wid * (n_tb * TB)

        def tok_src(u):
            s = u // n_tb
            t0 = t_base + (u % n_tb) * TB
            return tok_t.at[s, pl.ds(t0, TB)]

        def start_tok(u, tok_v, sem):
            pltpu.async_copy(tok_src(u), tok_v, sem)

        def wait_tok(u, tok_v, sem):
            pltpu.make_async_copy(tok_src(u), tok_v, sem).wait()

        def start_gather(idx_v, rows_v, sem):
            pltpu.async_copy(tabs.at[idx_v], rows_v, sem)

        def wait_gather(idx_v, rows_v, sem):
            pltpu.make_async_copy(tabs.at[idx_v], rows_v, sem).wait()

        def emit_unit(u, rows_v):
            s = u // n_tb
            t0 = t_base + (u % n_tb) * TB
            pltpu.sync_copy(rows_v, mid.at[pl.ds(t0, TB), s, :])

        pltpu.sync_copy(tok_src(0), tokA)
        start_gather(tokA, rowsA, gsA)
        start_tok(1, tokB, tsB)

        def pair(kk, carry):
            u0 = 2 * kk
            wait_tok(u0 + 1, tokB, tsB)
            wait_gather(tokA, rowsA, gsA)
            start_gather(tokB, rowsB, gsB)
            start_tok(u0 + 2, tokA, tsA)
            emit_unit(u0, rowsA)
            wait_tok(u0 + 2, tokA, tsA)
            wait_gather(tokB, rowsB, gsB)
            start_gather(tokA, rowsA, gsA)
            start_tok(u0 + 3, tokB, tsB)
            emit_unit(u0 + 1, rowsB)
            return carry

        lax.fori_loop(0, n_units // 2 - 1, pair, 0)

        wait_tok(n_units - 1, tokB, tsB)
        wait_gather(tokA, rowsA, gsA)
        start_gather(tokB, rowsB, gsB)
        emit_unit(n_units - 2, rowsA)
        wait_gather(tokB, rowsB, gsB)
        emit_unit(n_units - 1, rowsB)

    return k


def _out_transpose(S, T):
    def body(mid_ref, out_ref):
        m = mid_ref[0][:, :D]
        out_ref[0] = jnp.transpose(m, (1, 0))

    return pl.pallas_call(
        body,
        grid=(S, T // BT),
        in_specs=[pl.BlockSpec((1, BT, 2 * D), lambda s, tb: (s, tb, 0))],
        out_specs=pl.BlockSpec((1, D, BT), lambda s, tb: (s, 0, tb)),
        out_shape=jax.ShapeDtypeStruct((S, D, T), jnp.float32),
    )


def kernel(tokens, table):
    T, S = tokens.shape
    V = table.shape[0]
    tabs = _prep_table(V)(table.T)
    mid = _make_gather(T, S)(tokens.T, tabs)
    return mid[:, :, :D]


# speedup vs baseline: 2.3173x; 1.1053x over previous
import functools

import jax
import jax.numpy as jnp
from jax import lax
from jax.experimental import pallas as pl
from jax.experimental.pallas import tpu as pltpu
from jax.experimental.pallas import tpu_sc as plsc

D = 64
SCALE = 8.0
NC, NS, L = 2, 16, 16
NW = NC * NS
TB = 128
VB = 4096
BT = 512

_HIGH = jax.lax.Precision.HIGHEST


def _prep_table(V):
    def body(tt_ref, tabs_ref):
        e = jnp.where(
            lax.broadcasted_iota(jnp.int32, (D, 2 * D), 0)
            == lax.broadcasted_iota(jnp.int32, (D, 2 * D), 1),
            SCALE, 0.0).astype(jnp.bfloat16)
        x = tt_ref[...]
        hi = x.astype(jnp.bfloat16)
        r1 = x - hi.astype(jnp.float32)
        md = r1.astype(jnp.bfloat16)
        lo = (r1 - md.astype(jnp.float32)).astype(jnp.bfloat16)
        dims = (((0,), (0,)), ((), ()))
        acc = lax.dot_general(hi, e, dims,
                              preferred_element_type=jnp.float32)
        acc = acc + lax.dot_general(md, e, dims,
                                    preferred_element_type=jnp.float32)
        acc = acc + lax.dot_general(lo, e, dims,
                                    preferred_element_type=jnp.float32)
        tabs_ref[...] = acc

    return pl.pallas_call(
        body,
        grid=(pl.cdiv(V, VB),),
        in_specs=[pl.BlockSpec((D, VB), lambda v: (0, v))],
        out_specs=pl.BlockSpec((VB, 2 * D), lambda v: (v, 0)),
        out_shape=jax.ShapeDtypeStruct((V, 2 * D), jnp.float32),
    )


def _make_gather(T, S):
    n_tb = T // (NW * TB)
    n_units = S * n_tb
    assert n_units % 2 == 0 and n_units >= 4
    mesh = plsc.VectorSubcoreMesh(core_axis_name="c", subcore_axis_name="s")

    @functools.partial(
        pl.kernel,
        mesh=mesh,
        compiler_params=pltpu.CompilerParams(needs_layout_passes=False),
        out_type=jax.ShapeDtypeStruct((T, S, 2 * D), jnp.float32),
        scratch_types=[
            pltpu.VMEM((TB,), jnp.int32),
            pltpu.VMEM((TB,), jnp.int32),
            pltpu.VMEM((TB, 2 * D), jnp.float32),
            pltpu.VMEM((TB, 2 * D), jnp.float32),
            pltpu.SemaphoreType.DMA,
            pltpu.SemaphoreType.DMA,
            pltpu.SemaphoreType.DMA,
            pltpu.SemaphoreType.DMA,
        ],
    )
    def k(tok_t, tabs, mid, tokA, tokB, rowsA, rowsB, tsA, tsB, gsA, gsB):
        wid = lax.axis_index("s") * NC + lax.axis_index("c")
        t_base = wid * (n_tb * TB)

        def tok_src(u):
            s = u // n_tb
            t0 = t_base + (u % n_tb) * TB
            return tok_t.at[s, pl.ds(t0, TB)]

        def start_tok(u, tok_v, sem):
            pltpu.async_copy(tok_src(u), tok_v, sem)

        def wait_tok(u, tok_v, sem):
            pltpu.make_async_copy(tok_src(u), tok_v, sem).wait()

        def start_gather(idx_v, rows_v, sem):
            pltpu.async_copy(tabs.at[idx_v], rows_v, sem)

        def wait_gather(idx_v, rows_v, sem):
            pltpu.make_async_copy(tabs.at[idx_v], rows_v, sem).wait()

        def emit_unit(u, rows_v):
            s = u // n_tb
            t0 = t_base + (u % n_tb) * TB
            pltpu.sync_copy(rows_v, mid.at[pl.ds(t0, TB), s, :])

        pltpu.sync_copy(tok_src(0), tokA)
        start_gather(tokA, rowsA, gsA)
        start_tok(1, tokB, tsB)

        def pair(kk, carry):
            u0 = 2 * kk
            wait_tok(u0 + 1, tokB, tsB)
            wait_gather(tokA, rowsA, gsA)
            start_gather(tokB, rowsB, gsB)
            start_tok(u0 + 2, tokA, tsA)
            emit_unit(u0, rowsA)
            wait_tok(u0 + 2, tokA, tsA)
            wait_gather(tokB, rowsB, gsB)
            start_gather(tokA, rowsA, gsA)
            start_tok(u0 + 3, tokB, tsB)
            emit_unit(u0 + 1, rowsB)
            return carry

        lax.fori_loop(0, n_units // 2 - 1, pair, 0)

        wait_tok(n_units - 1, tokB, tsB)
        wait_gather(tokA, rowsA, gsA)
        start_gather(tokB, rowsB, gsB)
        emit_unit(n_units - 2, rowsA)
        wait_gather(tokB, rowsB, gsB)
        emit_unit(n_units - 1, rowsB)

    return k


def _out_transpose(S, T):
    def body(mid_ref, out_ref):
        m = mid_ref[0][:, :D]
        out_ref[0] = jnp.transpose(m, (1, 0))

    return pl.pallas_call(
        body,
        grid=(S, T // BT),
        in_specs=[pl.BlockSpec((1, BT, 2 * D), lambda s, tb: (s, tb, 0))],
        out_specs=pl.BlockSpec((1, D, BT), lambda s, tb: (s, 0, tb)),
        out_shape=jax.ShapeDtypeStruct((S, D, T), jnp.float32),
    )


def kernel(tokens, table):
    T, S = tokens.shape
    V = table.shape[0]
    tabs = _prep_table(V)(table.T)
    mid = _make_gather(T, S)(tokens.T, tabs)
    return mid[:, :, :D]


# speedup vs baseline: 2.5244x; 1.0894x over previous
import functools

import jax
import jax.numpy as jnp
from jax import lax
from jax.experimental import pallas as pl
from jax.experimental.pallas import tpu as pltpu
from jax.experimental.pallas import tpu_sc as plsc

D = 64
SCALE = 8.0
NC, NS, L = 2, 16, 16
NW = NC * NS
TB = 128
VB = 8192
BT = 512

_HIGH = jax.lax.Precision.HIGHEST


def _prep_table(V):
    def body(tt_ref, tabs_ref):
        e = jnp.where(
            lax.broadcasted_iota(jnp.int32, (D, 2 * D), 0)
            == lax.broadcasted_iota(jnp.int32, (D, 2 * D), 1),
            SCALE, 0.0).astype(jnp.bfloat16)
        x = tt_ref[...]
        hi = x.astype(jnp.bfloat16)
        r1 = x - hi.astype(jnp.float32)
        md = r1.astype(jnp.bfloat16)
        lo = (r1 - md.astype(jnp.float32)).astype(jnp.bfloat16)
        dims = (((0,), (0,)), ((), ()))
        acc = lax.dot_general(hi, e, dims,
                              preferred_element_type=jnp.float32)
        acc = acc + lax.dot_general(md, e, dims,
                                    preferred_element_type=jnp.float32)
        acc = acc + lax.dot_general(lo, e, dims,
                                    preferred_element_type=jnp.float32)
        tabs_ref[...] = acc

    return pl.pallas_call(
        body,
        grid=(pl.cdiv(V, VB),),
        in_specs=[pl.BlockSpec((D, VB), lambda v: (0, v))],
        out_specs=pl.BlockSpec((VB, 2 * D), lambda v: (v, 0)),
        out_shape=jax.ShapeDtypeStruct((V, 2 * D), jnp.float32),
    )


def _make_gather(T, S):
    n_tb = T // (NW * TB)
    n_units = S * n_tb
    assert n_units % 2 == 0 and n_units >= 4
    mesh = plsc.VectorSubcoreMesh(core_axis_name="c", subcore_axis_name="s")

    @functools.partial(
        pl.kernel,
        mesh=mesh,
        compiler_params=pltpu.CompilerParams(needs_layout_passes=False),
        out_type=jax.ShapeDtypeStruct((T, S, 2 * D), jnp.float32),
        scratch_types=[
            pltpu.VMEM((TB,), jnp.int32),
            pltpu.VMEM((TB,), jnp.int32),
            pltpu.VMEM((TB, 2 * D), jnp.float32),
            pltpu.VMEM((TB, 2 * D), jnp.float32),
            pltpu.SemaphoreType.DMA,
            pltpu.SemaphoreType.DMA,
            pltpu.SemaphoreType.DMA,
            pltpu.SemaphoreType.DMA,
        ],
    )
    def k(tok_t, tabs, mid, tokA, tokB, rowsA, rowsB, tsA, tsB, gsA, gsB):
        wid = lax.axis_index("s") * NC + lax.axis_index("c")
        t_base = wid * (n_tb * TB)

        def tok_src(u):
            s = u // n_tb
            t0 = t_base + (u % n_tb) * TB
            return tok_t.at[s, pl.ds(t0, TB)]

        def start_tok(u, tok_v, sem):
            pltpu.async_copy(tok_src(u), tok_v, sem)

        def wait_tok(u, tok_v, sem):
            pltpu.make_async_copy(tok_src(u), tok_v, sem).wait()

        def start_gather(idx_v, rows_v, sem):
            pltpu.async_copy(tabs.at[idx_v], rows_v, sem)

        def wait_gather(idx_v, rows_v, sem):
            pltpu.make_async_copy(tabs.at[idx_v], rows_v, sem).wait()

        def emit_unit(u, rows_v):
            s = u // n_tb
            t0 = t_base + (u % n_tb) * TB
            pltpu.sync_copy(rows_v, mid.at[pl.ds(t0, TB), s, :])

        pltpu.sync_copy(tok_src(0), tokA)
        start_gather(tokA, rowsA, gsA)
        start_tok(1, tokB, tsB)

        def pair(kk, carry):
            u0 = 2 * kk
            wait_tok(u0 + 1, tokB, tsB)
            wait_gather(tokA, rowsA, gsA)
            start_gather(tokB, rowsB, gsB)
            start_tok(u0 + 2, tokA, tsA)
            emit_unit(u0, rowsA)
            wait_tok(u0 + 2, tokA, tsA)
            wait_gather(tokB, rowsB, gsB)
            start_gather(tokA, rowsA, gsA)
            start_tok(u0 + 3, tokB, tsB)
            emit_unit(u0 + 1, rowsB)
            return carry

        lax.fori_loop(0, n_units // 2 - 1, pair, 0)

        wait_tok(n_units - 1, tokB, tsB)
        wait_gather(tokA, rowsA, gsA)
        start_gather(tokB, rowsB, gsB)
        emit_unit(n_units - 2, rowsA)
        wait_gather(tokB, rowsB, gsB)
        emit_unit(n_units - 1, rowsB)

    return k


def _out_transpose(S, T):
    def body(mid_ref, out_ref):
        m = mid_ref[0][:, :D]
        out_ref[0] = jnp.transpose(m, (1, 0))

    return pl.pallas_call(
        body,
        grid=(S, T // BT),
        in_specs=[pl.BlockSpec((1, BT, 2 * D), lambda s, tb: (s, tb, 0))],
        out_specs=pl.BlockSpec((1, D, BT), lambda s, tb: (s, 0, tb)),
        out_shape=jax.ShapeDtypeStruct((S, D, T), jnp.float32),
    )


def kernel(tokens, table):
    T, S = tokens.shape
    V = table.shape[0]
    tabs = _prep_table(V)(table.T)
    mid = _make_gather(T, S)(tokens.T, tabs)
    return mid[:, :, :D]


# speedup vs baseline: 2.6563x; 1.0522x over previous
import functools

import jax
import jax.numpy as jnp
from jax import lax
from jax.experimental import pallas as pl
from jax.experimental.pallas import tpu as pltpu
from jax.experimental.pallas import tpu_sc as plsc

D = 64
SCALE = 8.0
NC, NS, L = 2, 16, 16
NW = NC * NS
TB = 128
VB = 16384
BT = 512

_HIGH = jax.lax.Precision.HIGHEST


def _prep_table(V):
    def body(tt_ref, tabs_ref):
        e = jnp.where(
            lax.broadcasted_iota(jnp.int32, (D, 2 * D), 0)
            == lax.broadcasted_iota(jnp.int32, (D, 2 * D), 1),
            SCALE, 0.0).astype(jnp.bfloat16)
        x = tt_ref[...]
        hi = x.astype(jnp.bfloat16)
        r1 = x - hi.astype(jnp.float32)
        md = r1.astype(jnp.bfloat16)
        lo = (r1 - md.astype(jnp.float32)).astype(jnp.bfloat16)
        dims = (((0,), (0,)), ((), ()))
        acc = lax.dot_general(hi, e, dims,
                              preferred_element_type=jnp.float32)
        acc = acc + lax.dot_general(md, e, dims,
                                    preferred_element_type=jnp.float32)
        acc = acc + lax.dot_general(lo, e, dims,
                                    preferred_element_type=jnp.float32)
        tabs_ref[...] = acc

    return pl.pallas_call(
        body,
        grid=(pl.cdiv(V, VB),),
        in_specs=[pl.BlockSpec((D, VB), lambda v: (0, v))],
        out_specs=pl.BlockSpec((VB, 2 * D), lambda v: (v, 0)),
        out_shape=jax.ShapeDtypeStruct((V, 2 * D), jnp.float32),
    )


def _make_gather(T, S):
    n_tb = T // (NW * TB)
    n_units = S * n_tb
    assert n_units % 2 == 0 and n_units >= 4
    mesh = plsc.VectorSubcoreMesh(core_axis_name="c", subcore_axis_name="s")

    @functools.partial(
        pl.kernel,
        mesh=mesh,
        compiler_params=pltpu.CompilerParams(needs_layout_passes=False),
        out_type=jax.ShapeDtypeStruct((T, S, 2 * D), jnp.float32),
        scratch_types=[
            pltpu.VMEM((TB,), jnp.int32),
            pltpu.VMEM((TB,), jnp.int32),
            pltpu.VMEM((TB, 2 * D), jnp.float32),
            pltpu.VMEM((TB, 2 * D), jnp.float32),
            pltpu.SemaphoreType.DMA,
            pltpu.SemaphoreType.DMA,
            pltpu.SemaphoreType.DMA,
            pltpu.SemaphoreType.DMA,
        ],
    )
    def k(tok_t, tabs, mid, tokA, tokB, rowsA, rowsB, tsA, tsB, gsA, gsB):
        wid = lax.axis_index("s") * NC + lax.axis_index("c")
        t_base = wid * (n_tb * TB)

        def tok_src(u):
            s = u // n_tb
            t0 = t_base + (u % n_tb) * TB
            return tok_t.at[s, pl.ds(t0, TB)]

        def start_tok(u, tok_v, sem):
            pltpu.async_copy(tok_src(u), tok_v, sem)

        def wait_tok(u, tok_v, sem):
            pltpu.make_async_copy(tok_src(u), tok_v, sem).wait()

        def start_gather(idx_v, rows_v, sem):
            pltpu.async_copy(tabs.at[idx_v], rows_v, sem)

        def wait_gather(idx_v, rows_v, sem):
            pltpu.make_async_copy(tabs.at[idx_v], rows_v, sem).wait()

        def emit_unit(u, rows_v):
            s = u // n_tb
            t0 = t_base + (u % n_tb) * TB
            pltpu.sync_copy(rows_v, mid.at[pl.ds(t0, TB), s, :])

        pltpu.sync_copy(tok_src(0), tokA)
        start_gather(tokA, rowsA, gsA)
        start_tok(1, tokB, tsB)

        def pair(kk, carry):
            u0 = 2 * kk
            wait_tok(u0 + 1, tokB, tsB)
            wait_gather(tokA, rowsA, gsA)
            start_gather(tokB, rowsB, gsB)
            start_tok(u0 + 2, tokA, tsA)
            emit_unit(u0, rowsA)
            wait_tok(u0 + 2, tokA, tsA)
            wait_gather(tokB, rowsB, gsB)
            start_gather(tokA, rowsA, gsA)
            start_tok(u0 + 3, tokB, tsB)
            emit_unit(u0 + 1, rowsB)
            return carry

        lax.fori_loop(0, n_units // 2 - 1, pair, 0)

        wait_tok(n_units - 1, tokB, tsB)
        wait_gather(tokA, rowsA, gsA)
        start_gather(tokB, rowsB, gsB)
        emit_unit(n_units - 2, rowsA)
        wait_gather(tokB, rowsB, gsB)
        emit_unit(n_units - 1, rowsB)

    return k


def _out_transpose(S, T):
    def body(mid_ref, out_ref):
        m = mid_ref[0][:, :D]
        out_ref[0] = jnp.transpose(m, (1, 0))

    return pl.pallas_call(
        body,
        grid=(S, T // BT),
        in_specs=[pl.BlockSpec((1, BT, 2 * D), lambda s, tb: (s, tb, 0))],
        out_specs=pl.BlockSpec((1, D, BT), lambda s, tb: (s, 0, tb)),
        out_shape=jax.ShapeDtypeStruct((S, D, T), jnp.float32),
    )


def kernel(tokens, table):
    T, S = tokens.shape
    V = table.shape[0]
    tabs = _prep_table(V)(table.T)
    mid = _make_gather(T, S)(tokens.T, tabs)
    return mid[:, :, :D]


# speedup vs baseline: 2.6616x; 1.0020x over previous
import functools

import jax
import jax.numpy as jnp
from jax import lax
from jax.experimental import pallas as pl
from jax.experimental.pallas import tpu as pltpu
from jax.experimental.pallas import tpu_sc as plsc

D = 64
SCALE = 8.0
NC, NS, L = 2, 16, 16
NW = NC * NS
TB = 128
VB = 16384


def _prep_table(V):
    def body(tt_ref, tabs_ref):
        e = jnp.where(
            lax.broadcasted_iota(jnp.int32, (D, 2 * D), 0)
            == lax.broadcasted_iota(jnp.int32, (D, 2 * D), 1),
            SCALE, 0.0).astype(jnp.bfloat16)
        x = tt_ref[...]
        hi = x.astype(jnp.bfloat16)
        r1 = x - hi.astype(jnp.float32)
        md = r1.astype(jnp.bfloat16)
        lo = (r1 - md.astype(jnp.float32)).astype(jnp.bfloat16)
        dims = (((0,), (0,)), ((), ()))
        acc = lax.dot_general(hi, e, dims,
                              preferred_element_type=jnp.float32)
        acc = acc + lax.dot_general(md, e, dims,
                                    preferred_element_type=jnp.float32)
        acc = acc + lax.dot_general(lo, e, dims,
                                    preferred_element_type=jnp.float32)
        tabs_ref[...] = acc

    return pl.pallas_call(
        body,
        grid=(pl.cdiv(V, VB),),
        in_specs=[pl.BlockSpec((D, VB), lambda v: (0, v))],
        out_specs=pl.BlockSpec((VB, 2 * D), lambda v: (v, 0)),
        out_shape=jax.ShapeDtypeStruct((V, 2 * D), jnp.float32),
    )


def _make_gather(T, S):
    n_tb = T // (NW * TB)
    n_units = S * n_tb
    assert n_units % 2 == 0 and n_units >= 4
    mesh = plsc.VectorSubcoreMesh(core_axis_name="c", subcore_axis_name="s")

    @functools.partial(
        pl.kernel,
        mesh=mesh,
        compiler_params=pltpu.CompilerParams(needs_layout_passes=False),
        out_type=jax.ShapeDtypeStruct((T, S, 2 * D), jnp.float32),
        scratch_types=[
            pltpu.VMEM((TB,), jnp.int32),
            pltpu.VMEM((TB,), jnp.int32),
            pltpu.VMEM((TB, 2 * D), jnp.float32),
            pltpu.VMEM((TB, 2 * D), jnp.float32),
            pltpu.SemaphoreType.DMA,
            pltpu.SemaphoreType.DMA,
            pltpu.SemaphoreType.DMA,
            pltpu.SemaphoreType.DMA,
        ],
    )
    def k(tok_t, tabs, mid, tokA, tokB, rowsA, rowsB, tsA, tsB, gsA, gsB):
        wid = lax.axis_index("s") * NC + lax.axis_index("c")
        t_base = wid * (n_tb * TB)

        def tok_src(u):
            s = u // n_tb
            t0 = t_base + (u % n_tb) * TB
            return tok_t.at[s, pl.ds(t0, TB)]

        def start_tok(u, tok_v, sem):
            pltpu.async_copy(tok_src(u), tok_v, sem)

        def wait_tok(u, tok_v, sem):
            pltpu.make_async_copy(tok_src(u), tok_v, sem).wait()

        def start_gather(idx_v, rows_v, sem):
            pltpu.async_copy(tabs.at[idx_v], rows_v, sem)

        def wait_gather(idx_v, rows_v, sem):
            pltpu.make_async_copy(tabs.at[idx_v], rows_v, sem).wait()

        def emit_unit(u, rows_v):
            s = u // n_tb
            t0 = t_base + (u % n_tb) * TB
            pltpu.sync_copy(rows_v, mid.at[pl.ds(t0, TB), s, :])

        pltpu.sync_copy(tok_src(0), tokA)
        start_gather(tokA, rowsA, gsA)
        start_tok(1, tokB, tsB)

        def pair(kk, carry):
            u0 = 2 * kk
            wait_tok(u0 + 1, tokB, tsB)
            wait_gather(tokA, rowsA, gsA)
            start_gather(tokB, rowsB, gsB)
            start_tok(u0 + 2, tokA, tsA)
            emit_unit(u0, rowsA)
            wait_tok(u0 + 2, tokA, tsA)
            wait_gather(tokB, rowsB, gsB)
            start_gather(tokA, rowsA, gsA)
            start_tok(u0 + 3, tokB, tsB)
            emit_unit(u0 + 1, rowsB)
            return carry

        lax.fori_loop(0, n_units // 2 - 1, pair, 0)

        wait_tok(n_units - 1, tokB, tsB)
        wait_gather(tokA, rowsA, gsA)
        start_gather(tokB, rowsB, gsB)
        emit_unit(n_units - 2, rowsA)
        wait_gather(tokB, rowsB, gsB)
        emit_unit(n_units - 1, rowsB)

    return k


def kernel(tokens, table):
    T, S = tokens.shape
    V = table.shape[0]
    tabs = _prep_table(V)(table.T)
    mid = _make_gather(T, S)(tokens.T, tabs)
    return mid[:, :, :D]
